# baseline (speedup 1.0000x reference)
import jax
import jax.numpy as jnp
from jax.experimental import pallas as pl
from jax.experimental.pallas import tpu as pltpu


def _body(x_ref, out_ref, acc_ref):
    acc_ref[0, 0, 0, 0] = jnp.sum(x_ref[...])
    out_ref[...] = x_ref[...]


def kernel(x, evaluate_tables, focus_tables):
    B, C, H, W = x.shape
    out, _ = pl.pallas_call(
        _body,
        grid=(B,),
        in_specs=[pl.BlockSpec((1, C, H, W), lambda b: (b, 0, 0, 0))],
        out_specs=[
            pl.BlockSpec((1, C, H, W), lambda b: (b, 0, 0, 0)),
            pl.BlockSpec((1, 1, 1, 1), lambda b: (b, 0, 0, 0),
                         memory_space=pltpu.SMEM),
        ],
        out_shape=[
            jax.ShapeDtypeStruct((B, C, H, W), x.dtype),
            jax.ShapeDtypeStruct((B, 1, 1, 1), jnp.float32),
        ],
        compiler_params=pltpu.CompilerParams(
            dimension_semantics=("parallel",),
        ),
    )(x)
    return out
